# Initial kernel scaffold; baseline (speedup 1.0000x reference)
import jax
import jax.numpy as jnp
from jax.experimental import pallas as pl


def kernel(features, ground_truth):
    raise NotImplementedError("write your pallas kernel here")


# speedup vs baseline: 21.8918x; 21.8918x over previous
import jax
import jax.numpy as jnp
from jax.experimental import pallas as pl

_DELTA_V = 0.2
_DELTA_D = 0.2
_ALPHA = 1.0
_BETA = 1.0
_GAMMA = 0.001
_K = 8


def _cluster_kernel(f_ref, gt_ref, mu_ref, scal_ref):
    f = f_ref[0]
    g = gt_ref[0]
    C, P = f.shape
    K = _K

    kiota = jax.lax.broadcasted_iota(jnp.int32, (K, P), 0)
    mask = (kiota == g).astype(jnp.float32)
    counts = jnp.sum(mask, axis=1, keepdims=True)

    sums = jax.lax.dot_general(
        mask, f, (((1,), (1,)), ((), ())),
        preferred_element_type=jnp.float32)
    mu = sums / counts

    q = jax.lax.dot_general(
        mu, f, (((1,), (0,)), ((), ())),
        preferred_element_type=jnp.float32)
    fsq = jnp.sum(f * f, axis=0, keepdims=True)
    musq = jnp.sum(mu * mu, axis=1, keepdims=True)
    dsel = jnp.sum(mask * q, axis=0, keepdims=True)
    msel = jnp.sum(mask * musq, axis=0, keepdims=True)
    dist = jnp.sqrt(jnp.maximum(fsq - 2.0 * dsel + msel, 0.0))

    pk = jnp.sum(mask * dist, axis=1, keepdims=True)
    per_k = pk / counts
    hinge = jnp.maximum(per_k - _DELTA_V, 0.0)
    inv = 1.0 / counts
    local_var = jnp.sum(hinge * inv) / jnp.sum(inv)

    gram = jax.lax.dot_general(
        mu, mu, (((1,), (1,)), ((), ())),
        preferred_element_type=jnp.float32)
    dsq = musq + jnp.transpose(musq) - 2.0 * gram
    ii = jax.lax.broadcasted_iota(jnp.int32, (K, K), 0)
    jj = jax.lax.broadcasted_iota(jnp.int32, (K, K), 1)
    eye = (ii == jj).astype(jnp.float32)
    dmat = jnp.sqrt(jnp.maximum(dsq, 0.0) + eye) * (1.0 - eye)
    mean_dist = jnp.sum(dmat, axis=1, keepdims=True) / (K - 1)
    dloss = jnp.sum(jnp.maximum(2.0 * _DELTA_D - mean_dist, 0.0))
    nloss = jnp.sum(jnp.sqrt(musq)) / K

    mu_ref[0] = mu
    lane = jax.lax.broadcasted_iota(jnp.int32, (1, 128), 1)
    scal = (jnp.where(lane == 0, local_var, 0.0)
            + jnp.where(lane == 1, dloss, 0.0)
            + jnp.where(lane == 2, nloss, 0.0))
    scal_ref[0] = scal


def kernel(features, ground_truth):
    N, C, H, W = features.shape
    P = H * W
    f = features.reshape(N, C, P)
    gt = ground_truth.reshape(N, 1, P)

    mu_kc, scal = pl.pallas_call(
        _cluster_kernel,
        grid=(N,),
        in_specs=[
            pl.BlockSpec((1, C, P), lambda n: (n, 0, 0)),
            pl.BlockSpec((1, 1, P), lambda n: (n, 0, 0)),
        ],
        out_specs=[
            pl.BlockSpec((1, _K, C), lambda n: (n, 0, 0)),
            pl.BlockSpec((1, 1, 128), lambda n: (n, 0, 0)),
        ],
        out_shape=[
            jax.ShapeDtypeStruct((N, _K, C), jnp.float32),
            jax.ShapeDtypeStruct((N, 1, 128), jnp.float32),
        ],
    )(f, gt)

    mu = jnp.transpose(mu_kc, (0, 2, 1))
    local_var = scal[:, 0, 0]
    dl = scal[:, 0, 1]
    nl = scal[:, 0, 2]
    variance_loss = jnp.mean(local_var)
    distance_loss = jnp.sum(dl) / (N * _K)
    normalization_loss = jnp.sum(nl) / N
    total = (_ALPHA * variance_loss + _BETA * distance_loss
             + _GAMMA * normalization_loss)
    return total, variance_loss, distance_loss, normalization_loss, mu


# speedup vs baseline: 23.3341x; 1.0659x over previous
import jax
import jax.numpy as jnp
from jax.experimental import pallas as pl

_DELTA_V = 0.2
_DELTA_D = 0.2
_ALPHA = 1.0
_BETA = 1.0
_GAMMA = 0.001
_K = 8


def _cluster_kernel(f_ref, gt_ref, mu_ref, scal_ref):
    n = pl.program_id(0)
    nsteps = pl.num_programs(0)
    f = f_ref[0]
    g = gt_ref[0]
    C, P = f.shape
    K = _K

    kiota = jax.lax.broadcasted_iota(jnp.int32, (K, P), 0)
    mask = (kiota == g).astype(jnp.float32)
    counts = jnp.sum(mask, axis=1, keepdims=True)

    sums = jax.lax.dot_general(
        mask, f, (((1,), (1,)), ((), ())),
        preferred_element_type=jnp.float32)
    mu = sums / counts

    q = jax.lax.dot_general(
        mu, f, (((1,), (0,)), ((), ())),
        preferred_element_type=jnp.float32)
    ones_c = jnp.ones((1, C), dtype=jnp.float32)
    fsq = jax.lax.dot_general(
        ones_c, f * f, (((1,), (0,)), ((), ())),
        preferred_element_type=jnp.float32)
    musq = jnp.sum(mu * mu, axis=1, keepdims=True)
    tsel = jnp.sum(mask * (musq - 2.0 * q), axis=0,
                   keepdims=True)
    dist = jnp.sqrt(jnp.maximum(fsq + tsel, 0.0))

    pk = jax.lax.dot_general(
        mask, dist, (((1,), (1,)), ((), ())),
        preferred_element_type=jnp.float32)
    per_k = pk / counts
    hinge = jnp.maximum(per_k - _DELTA_V, 0.0)
    inv = 1.0 / counts
    local_var = jnp.sum(hinge * inv) / jnp.sum(inv)

    gram = jax.lax.dot_general(
        mu, mu, (((1,), (1,)), ((), ())),
        preferred_element_type=jnp.float32)
    dsq = musq + jnp.transpose(musq) - 2.0 * gram
    ii = jax.lax.broadcasted_iota(jnp.int32, (K, K), 0)
    jj = jax.lax.broadcasted_iota(jnp.int32, (K, K), 1)
    eye = (ii == jj).astype(jnp.float32)
    dmat = jnp.sqrt(jnp.maximum(dsq, 0.0) + eye) * (1.0 - eye)
    mean_dist = jnp.sum(dmat, axis=1, keepdims=True) / (K - 1)
    dloss = jnp.sum(jnp.maximum(2.0 * _DELTA_D - mean_dist, 0.0))
    nloss = jnp.sum(jnp.sqrt(musq)) / K

    mu_ref[0] = jnp.transpose(mu)

    lane = jax.lax.broadcasted_iota(jnp.int32, (1, 128), 1)
    vec = (jnp.where(lane == 8, local_var, 0.0)
           + jnp.where(lane == 9, dloss, 0.0)
           + jnp.where(lane == 10, nloss, 0.0))

    @pl.when(n == 0)
    def _init():
        scal_ref[0] = vec

    @pl.when(n > 0)
    def _acc():
        scal_ref[0] = scal_ref[0] + vec

    @pl.when(n == nsteps - 1)
    def _finalize():
        acc = scal_ref[0]
        s_var = jnp.sum(jnp.where(lane == 8, acc, 0.0))
        s_d = jnp.sum(jnp.where(lane == 9, acc, 0.0))
        s_n = jnp.sum(jnp.where(lane == 10, acc, 0.0))
        fn = jnp.float32(nsteps)
        variance_loss = s_var / fn
        distance_loss = s_d / (fn * K)
        normalization_loss = s_n / fn
        total = (_ALPHA * variance_loss + _BETA * distance_loss
                 + _GAMMA * normalization_loss)
        scal_ref[0] = (acc
                       + jnp.where(lane == 0, total, 0.0)
                       + jnp.where(lane == 1, variance_loss, 0.0)
                       + jnp.where(lane == 2, distance_loss, 0.0)
                       + jnp.where(lane == 3, normalization_loss, 0.0))


def kernel(features, ground_truth):
    N, C, H, W = features.shape
    P = H * W
    f = features.reshape(N, C, P)
    gt = ground_truth.reshape(N, 1, P)

    mu, scal = pl.pallas_call(
        _cluster_kernel,
        grid=(N,),
        in_specs=[
            pl.BlockSpec((1, C, P), lambda n: (n, 0, 0)),
            pl.BlockSpec((1, 1, P), lambda n: (n, 0, 0)),
        ],
        out_specs=[
            pl.BlockSpec((1, C, _K), lambda n: (n, 0, 0)),
            pl.BlockSpec((1, 1, 128), lambda n: (0, 0, 0)),
        ],
        out_shape=[
            jax.ShapeDtypeStruct((N, C, _K), jnp.float32),
            jax.ShapeDtypeStruct((1, 1, 128), jnp.float32),
        ],
    )(f, gt)

    s = scal.reshape(128)
    total = s[0]
    variance_loss = s[1]
    distance_loss = s[2]
    normalization_loss = s[3]
    return total, variance_loss, distance_loss, normalization_loss, mu


# speedup vs baseline: 24.1472x; 1.0348x over previous
import jax
import jax.numpy as jnp
from jax.experimental import pallas as pl
from jax.experimental.pallas import tpu as pltpu

_DELTA_V = 0.2
_DELTA_D = 0.2
_ALPHA = 1.0
_BETA = 1.0
_GAMMA = 0.001
_K = 8


def _cluster_kernel(f_ref, gt_ref, mu_ref, tot_ref, var_ref, dst_ref,
                    nrm_ref, acc_ref):
    n = pl.program_id(0)
    nsteps = pl.num_programs(0)
    f = f_ref[0]
    g = gt_ref[0]
    C, P = f.shape
    K = _K

    kiota = jax.lax.broadcasted_iota(jnp.int32, (K, P), 0)
    mask = (kiota == g).astype(jnp.float32)
    counts = jnp.sum(mask, axis=1, keepdims=True)

    sums = jax.lax.dot_general(
        mask, f, (((1,), (1,)), ((), ())),
        preferred_element_type=jnp.float32)
    mu = sums / counts

    q = jax.lax.dot_general(
        mu, f, (((1,), (0,)), ((), ())),
        preferred_element_type=jnp.float32)
    ones_c = jnp.ones((1, C), dtype=jnp.float32)
    fsq = jax.lax.dot_general(
        ones_c, f * f, (((1,), (0,)), ((), ())),
        preferred_element_type=jnp.float32)
    musq = jnp.sum(mu * mu, axis=1, keepdims=True)
    tsel = jnp.sum(mask * (musq - 2.0 * q), axis=0,
                   keepdims=True)
    dist = jnp.sqrt(jnp.maximum(fsq + tsel, 0.0))

    pk = jax.lax.dot_general(
        mask, dist, (((1,), (1,)), ((), ())),
        preferred_element_type=jnp.float32)
    per_k = pk / counts
    hinge = jnp.maximum(per_k - _DELTA_V, 0.0)
    inv = 1.0 / counts
    local_var = jnp.sum(hinge * inv) / jnp.sum(inv)

    gram = jax.lax.dot_general(
        mu, mu, (((1,), (1,)), ((), ())),
        preferred_element_type=jnp.float32)
    dsq = musq + jnp.transpose(musq) - 2.0 * gram
    ii = jax.lax.broadcasted_iota(jnp.int32, (K, K), 0)
    jj = jax.lax.broadcasted_iota(jnp.int32, (K, K), 1)
    eye = (ii == jj).astype(jnp.float32)
    dmat = jnp.sqrt(jnp.maximum(dsq, 0.0) + eye) * (1.0 - eye)
    mean_dist = jnp.sum(dmat, axis=1, keepdims=True) / (K - 1)
    dloss = jnp.sum(jnp.maximum(2.0 * _DELTA_D - mean_dist, 0.0))
    nloss = jnp.sum(jnp.sqrt(musq)) / K

    mu_ref[0] = jnp.transpose(mu)

    lane = jax.lax.broadcasted_iota(jnp.int32, (1, 128), 1)
    vec = (jnp.where(lane == 8, local_var, 0.0)
           + jnp.where(lane == 9, dloss, 0.0)
           + jnp.where(lane == 10, nloss, 0.0))

    @pl.when(n == 0)
    def _init():
        acc_ref[...] = vec

    @pl.when(n > 0)
    def _acc():
        acc_ref[...] = acc_ref[...] + vec

    @pl.when(n == nsteps - 1)
    def _finalize():
        acc = acc_ref[...]
        s_var = jnp.sum(jnp.where(lane == 8, acc, 0.0))
        s_d = jnp.sum(jnp.where(lane == 9, acc, 0.0))
        s_n = jnp.sum(jnp.where(lane == 10, acc, 0.0))
        fn = jnp.float32(nsteps)
        variance_loss = s_var / fn
        distance_loss = s_d / (fn * K)
        normalization_loss = s_n / fn
        total = (_ALPHA * variance_loss + _BETA * distance_loss
                 + _GAMMA * normalization_loss)
        tot_ref[0] = jnp.full((1, 1), total, jnp.float32)
        var_ref[0] = jnp.full((1, 1), variance_loss, jnp.float32)
        dst_ref[0] = jnp.full((1, 1), distance_loss, jnp.float32)
        nrm_ref[0] = jnp.full((1, 1), normalization_loss, jnp.float32)


def kernel(features, ground_truth):
    N, C, H, W = features.shape
    P = H * W
    f = features.reshape(N, C, P)
    gt = ground_truth.reshape(N, 1, P)

    scalar_spec = pl.BlockSpec((1, 1, 1), lambda n: (0, 0, 0))
    scalar_shape = jax.ShapeDtypeStruct((1, 1, 1), jnp.float32)
    mu, tot, var, dst, nrm = pl.pallas_call(
        _cluster_kernel,
        grid=(N,),
        in_specs=[
            pl.BlockSpec((1, C, P), lambda n: (n, 0, 0)),
            pl.BlockSpec((1, 1, P), lambda n: (n, 0, 0)),
        ],
        out_specs=[
            pl.BlockSpec((1, C, _K), lambda n: (n, 0, 0)),
            scalar_spec, scalar_spec, scalar_spec, scalar_spec,
        ],
        out_shape=[
            jax.ShapeDtypeStruct((N, C, _K), jnp.float32),
            scalar_shape, scalar_shape, scalar_shape, scalar_shape,
        ],
        scratch_shapes=[pltpu.VMEM((1, 128), jnp.float32)],
    )(f, gt)

    return (tot.reshape(()), var.reshape(()), dst.reshape(()),
            nrm.reshape(()), mu)


# speedup vs baseline: 26.2764x; 1.0882x over previous
import jax
import jax.numpy as jnp
from jax.experimental import pallas as pl
from jax.experimental.pallas import tpu as pltpu

_DELTA_V = 0.2
_DELTA_D = 0.2
_ALPHA = 1.0
_BETA = 1.0
_GAMMA = 0.001
_K = 8


def _cluster_kernel(f_ref, gt_ref, mu_ref, tot_ref, var_ref, dst_ref,
                    nrm_ref, acc_ref):
    n = pl.program_id(0)
    nsteps = pl.num_programs(0)
    f = f_ref[0]
    g = gt_ref[0]
    C, P = f.shape
    K = _K

    kiota = jax.lax.broadcasted_iota(jnp.int32, (K, P), 0)
    mask = (kiota == g).astype(jnp.float32)
    counts = jnp.sum(mask, axis=1, keepdims=True)

    sums = jax.lax.dot_general(
        mask, f, (((1,), (1,)), ((), ())),
        preferred_element_type=jnp.float32)
    mu = sums / counts

    q = jax.lax.dot_general(
        mu, f, (((1,), (0,)), ((), ())),
        preferred_element_type=jnp.float32)
    fsq = jnp.sum(f * f, axis=0, keepdims=True)
    musq = jnp.sum(mu * mu, axis=1, keepdims=True)
    tsel = jnp.sum(mask * (musq - 2.0 * q), axis=0,
                   keepdims=True)
    dist = jnp.sqrt(jnp.maximum(fsq + tsel, 0.0))

    pk = jnp.sum(mask * dist, axis=1, keepdims=True)
    per_k = pk / counts
    hinge = jnp.maximum(per_k - _DELTA_V, 0.0)
    inv = 1.0 / counts
    local_var = jnp.sum(hinge * inv) / jnp.sum(inv)

    gram = jax.lax.dot_general(
        mu, mu, (((1,), (1,)), ((), ())),
        preferred_element_type=jnp.float32)
    dsq = musq + jnp.transpose(musq) - 2.0 * gram
    ii = jax.lax.broadcasted_iota(jnp.int32, (K, K), 0)
    jj = jax.lax.broadcasted_iota(jnp.int32, (K, K), 1)
    eye = (ii == jj).astype(jnp.float32)
    dmat = jnp.sqrt(jnp.maximum(dsq, 0.0) + eye) * (1.0 - eye)
    mean_dist = jnp.sum(dmat, axis=1, keepdims=True) / (K - 1)
    dloss = jnp.sum(jnp.maximum(2.0 * _DELTA_D - mean_dist, 0.0))
    nloss = jnp.sum(jnp.sqrt(musq)) / K

    mu_ref[0] = jnp.transpose(mu)

    lane = jax.lax.broadcasted_iota(jnp.int32, (1, 128), 1)
    vec = (jnp.where(lane == 8, local_var, 0.0)
           + jnp.where(lane == 9, dloss, 0.0)
           + jnp.where(lane == 10, nloss, 0.0))

    @pl.when(n == 0)
    def _init():
        acc_ref[...] = vec

    @pl.when(n > 0)
    def _acc():
        acc_ref[...] = acc_ref[...] + vec

    @pl.when(n == nsteps - 1)
    def _finalize():
        acc = acc_ref[...]
        s_var = jnp.sum(jnp.where(lane == 8, acc, 0.0))
        s_d = jnp.sum(jnp.where(lane == 9, acc, 0.0))
        s_n = jnp.sum(jnp.where(lane == 10, acc, 0.0))
        fn = jnp.float32(nsteps)
        variance_loss = s_var / fn
        distance_loss = s_d / (fn * K)
        normalization_loss = s_n / fn
        total = (_ALPHA * variance_loss + _BETA * distance_loss
                 + _GAMMA * normalization_loss)
        tot_ref[0] = jnp.full((1, 1), total, jnp.float32)
        var_ref[0] = jnp.full((1, 1), variance_loss, jnp.float32)
        dst_ref[0] = jnp.full((1, 1), distance_loss, jnp.float32)
        nrm_ref[0] = jnp.full((1, 1), normalization_loss, jnp.float32)


def kernel(features, ground_truth):
    N, C, H, W = features.shape
    P = H * W
    f = features.reshape(N, C, P)
    gt = ground_truth.reshape(N, 1, P)

    scalar_spec = pl.BlockSpec((1, 1, 1), lambda n: (0, 0, 0))
    scalar_shape = jax.ShapeDtypeStruct((1, 1, 1), jnp.float32)
    mu, tot, var, dst, nrm = pl.pallas_call(
        _cluster_kernel,
        grid=(N,),
        in_specs=[
            pl.BlockSpec((1, C, P), lambda n: (n, 0, 0)),
            pl.BlockSpec((1, 1, P), lambda n: (n, 0, 0)),
        ],
        out_specs=[
            pl.BlockSpec((1, C, _K), lambda n: (n, 0, 0)),
            scalar_spec, scalar_spec, scalar_spec, scalar_spec,
        ],
        out_shape=[
            jax.ShapeDtypeStruct((N, C, _K), jnp.float32),
            scalar_shape, scalar_shape, scalar_shape, scalar_shape,
        ],
        scratch_shapes=[pltpu.VMEM((1, 128), jnp.float32)],
    )(f, gt)

    return (tot.reshape(()), var.reshape(()), dst.reshape(()),
            nrm.reshape(()), mu)


# speedup vs baseline: 55.3335x; 2.1058x over previous
import jax
import jax.numpy as jnp
from jax.experimental import pallas as pl
from jax.experimental.pallas import tpu as pltpu

_DELTA_V = 0.2
_DELTA_D = 0.2
_ALPHA = 1.0
_BETA = 1.0
_GAMMA = 0.001
_K = 8


def _cluster_kernel(f_ref, gt_ref, mu_ref, tot_ref, var_ref, dst_ref,
                    nrm_ref, acc_ref):
    n = pl.program_id(0)
    nsteps = pl.num_programs(0)
    f4 = f_ref[0]
    g2 = gt_ref[0]
    C = f4.shape[0]
    P = f4.shape[1] * f4.shape[2]
    f = f4.reshape(C, P)
    g = g2.reshape(1, P)
    K = _K

    kiota = jax.lax.broadcasted_iota(jnp.int32, (K, P), 0)
    mask = (kiota == g).astype(jnp.float32)
    counts = jnp.sum(mask, axis=1, keepdims=True)

    sums = jax.lax.dot_general(
        mask, f, (((1,), (1,)), ((), ())),
        preferred_element_type=jnp.float32)
    mu = sums / counts

    q = jax.lax.dot_general(
        mu, f, (((1,), (0,)), ((), ())),
        preferred_element_type=jnp.float32)
    fsq = jnp.sum(f * f, axis=0, keepdims=True)
    musq = jnp.sum(mu * mu, axis=1, keepdims=True)
    tsel = jnp.sum(mask * (musq - 2.0 * q), axis=0,
                   keepdims=True)
    dist = jnp.sqrt(jnp.maximum(fsq + tsel, 0.0))

    pk = jnp.sum(mask * dist, axis=1, keepdims=True)
    per_k = pk / counts
    hinge = jnp.maximum(per_k - _DELTA_V, 0.0)
    inv = 1.0 / counts
    local_var = jnp.sum(hinge * inv) / jnp.sum(inv)

    gram = jax.lax.dot_general(
        mu, mu, (((1,), (1,)), ((), ())),
        preferred_element_type=jnp.float32)
    dsq = musq + jnp.transpose(musq) - 2.0 * gram
    ii = jax.lax.broadcasted_iota(jnp.int32, (K, K), 0)
    jj = jax.lax.broadcasted_iota(jnp.int32, (K, K), 1)
    eye = (ii == jj).astype(jnp.float32)
    dmat = jnp.sqrt(jnp.maximum(dsq, 0.0) + eye) * (1.0 - eye)
    mean_dist = jnp.sum(dmat, axis=1, keepdims=True) / (K - 1)
    dloss = jnp.sum(jnp.maximum(2.0 * _DELTA_D - mean_dist, 0.0))
    nloss = jnp.sum(jnp.sqrt(musq)) / K

    mu_ref[0] = jnp.transpose(mu)

    lane = jax.lax.broadcasted_iota(jnp.int32, (1, 128), 1)
    vec = (jnp.where(lane == 8, local_var, 0.0)
           + jnp.where(lane == 9, dloss, 0.0)
           + jnp.where(lane == 10, nloss, 0.0))

    @pl.when(n == 0)
    def _init():
        acc_ref[...] = vec

    @pl.when(n > 0)
    def _acc():
        acc_ref[...] = acc_ref[...] + vec

    @pl.when(n == nsteps - 1)
    def _finalize():
        acc = acc_ref[...]
        s_var = jnp.sum(jnp.where(lane == 8, acc, 0.0))
        s_d = jnp.sum(jnp.where(lane == 9, acc, 0.0))
        s_n = jnp.sum(jnp.where(lane == 10, acc, 0.0))
        fn = jnp.float32(nsteps)
        variance_loss = s_var / fn
        distance_loss = s_d / (fn * K)
        normalization_loss = s_n / fn
        total = (_ALPHA * variance_loss + _BETA * distance_loss
                 + _GAMMA * normalization_loss)
        tot_ref[0] = jnp.full((1, 1), total, jnp.float32)
        var_ref[0] = jnp.full((1, 1), variance_loss, jnp.float32)
        dst_ref[0] = jnp.full((1, 1), distance_loss, jnp.float32)
        nrm_ref[0] = jnp.full((1, 1), normalization_loss, jnp.float32)


def kernel(features, ground_truth):
    N, C, H, W = features.shape

    scalar_spec = pl.BlockSpec((1, 1, 1), lambda n: (0, 0, 0))
    scalar_shape = jax.ShapeDtypeStruct((1, 1, 1), jnp.float32)
    mu, tot, var, dst, nrm = pl.pallas_call(
        _cluster_kernel,
        grid=(N,),
        in_specs=[
            pl.BlockSpec((1, C, H, W), lambda n: (n, 0, 0, 0)),
            pl.BlockSpec((1, H, W), lambda n: (n, 0, 0)),
        ],
        out_specs=[
            pl.BlockSpec((1, C, _K), lambda n: (n, 0, 0)),
            scalar_spec, scalar_spec, scalar_spec, scalar_spec,
        ],
        out_shape=[
            jax.ShapeDtypeStruct((N, C, _K), jnp.float32),
            scalar_shape, scalar_shape, scalar_shape, scalar_shape,
        ],
        scratch_shapes=[pltpu.VMEM((1, 128), jnp.float32)],
    )(features, ground_truth)

    return (tot.reshape(()), var.reshape(()), dst.reshape(()),
            nrm.reshape(()), mu)


# speedup vs baseline: 59.4599x; 1.0746x over previous
import jax
import jax.numpy as jnp
from jax.experimental import pallas as pl
from jax.experimental.pallas import tpu as pltpu

_DELTA_V = 0.2
_DELTA_D = 0.2
_ALPHA = 1.0
_BETA = 1.0
_GAMMA = 0.001
_K = 8


def _cluster_kernel(f_ref, gt_ref, mu_ref, tot_ref, var_ref, dst_ref,
                    nrm_ref, acc_ref):
    n = pl.program_id(0)
    nsteps = pl.num_programs(0)
    f4 = f_ref[0]
    g2 = gt_ref[0]
    C = f4.shape[0]
    P = f4.shape[1] * f4.shape[2]
    fb = f4.astype(jnp.bfloat16).reshape(C, P)
    g = g2.reshape(1, P)
    K = _K

    kiota = jax.lax.broadcasted_iota(jnp.int32, (K, P), 0)
    mask = (kiota == g).astype(jnp.float32)
    mask_b = mask.astype(jnp.bfloat16)
    counts = jnp.sum(mask, axis=1, keepdims=True)

    sums = jax.lax.dot_general(
        mask_b, fb, (((1,), (1,)), ((), ())),
        preferred_element_type=jnp.float32)
    mu = sums / counts

    q = jax.lax.dot_general(
        mu.astype(jnp.bfloat16), fb, (((1,), (0,)), ((), ())),
        preferred_element_type=jnp.float32)
    fsq = jnp.sum(f4 * f4, axis=0).reshape(1, P)
    musq = jnp.sum(mu * mu, axis=1, keepdims=True)
    tsel = jnp.sum(mask * (musq - 2.0 * q), axis=0,
                   keepdims=True)
    dist = jnp.sqrt(jnp.maximum(fsq + tsel, 0.0))

    pk = jnp.sum(mask * dist, axis=1, keepdims=True)
    per_k = pk / counts
    hinge = jnp.maximum(per_k - _DELTA_V, 0.0)
    inv = 1.0 / counts
    local_var = jnp.sum(hinge * inv) / jnp.sum(inv)

    gram = jax.lax.dot_general(
        mu, mu, (((1,), (1,)), ((), ())),
        preferred_element_type=jnp.float32)
    dsq = musq + jnp.transpose(musq) - 2.0 * gram
    ii = jax.lax.broadcasted_iota(jnp.int32, (K, K), 0)
    jj = jax.lax.broadcasted_iota(jnp.int32, (K, K), 1)
    eye = (ii == jj).astype(jnp.float32)
    dmat = jnp.sqrt(jnp.maximum(dsq, 0.0) + eye) * (1.0 - eye)
    mean_dist = jnp.sum(dmat, axis=1, keepdims=True) / (K - 1)
    dloss = jnp.sum(jnp.maximum(2.0 * _DELTA_D - mean_dist, 0.0))
    nloss = jnp.sum(jnp.sqrt(musq)) / K

    mu_ref[0] = jnp.transpose(mu)

    lane = jax.lax.broadcasted_iota(jnp.int32, (1, 128), 1)
    vec = (jnp.where(lane == 8, local_var, 0.0)
           + jnp.where(lane == 9, dloss, 0.0)
           + jnp.where(lane == 10, nloss, 0.0))

    @pl.when(n == 0)
    def _init():
        acc_ref[...] = vec

    @pl.when(n > 0)
    def _acc():
        acc_ref[...] = acc_ref[...] + vec

    @pl.when(n == nsteps - 1)
    def _finalize():
        acc = acc_ref[...]
        s_var = jnp.sum(jnp.where(lane == 8, acc, 0.0))
        s_d = jnp.sum(jnp.where(lane == 9, acc, 0.0))
        s_n = jnp.sum(jnp.where(lane == 10, acc, 0.0))
        fn = jnp.float32(nsteps)
        variance_loss = s_var / fn
        distance_loss = s_d / (fn * K)
        normalization_loss = s_n / fn
        total = (_ALPHA * variance_loss + _BETA * distance_loss
                 + _GAMMA * normalization_loss)
        tot_ref[0] = jnp.full((1, 1), total, jnp.float32)
        var_ref[0] = jnp.full((1, 1), variance_loss, jnp.float32)
        dst_ref[0] = jnp.full((1, 1), distance_loss, jnp.float32)
        nrm_ref[0] = jnp.full((1, 1), normalization_loss, jnp.float32)


def kernel(features, ground_truth):
    N, C, H, W = features.shape

    scalar_spec = pl.BlockSpec((1, 1, 1), lambda n: (0, 0, 0))
    scalar_shape = jax.ShapeDtypeStruct((1, 1, 1), jnp.float32)
    mu, tot, var, dst, nrm = pl.pallas_call(
        _cluster_kernel,
        grid=(N,),
        in_specs=[
            pl.BlockSpec((1, C, H, W), lambda n: (n, 0, 0, 0)),
            pl.BlockSpec((1, H, W), lambda n: (n, 0, 0)),
        ],
        out_specs=[
            pl.BlockSpec((1, C, _K), lambda n: (n, 0, 0)),
            scalar_spec, scalar_spec, scalar_spec, scalar_spec,
        ],
        out_shape=[
            jax.ShapeDtypeStruct((N, C, _K), jnp.float32),
            scalar_shape, scalar_shape, scalar_shape, scalar_shape,
        ],
        scratch_shapes=[pltpu.VMEM((1, 128), jnp.float32)],
    )(features, ground_truth)

    return (tot.reshape(()), var.reshape(()), dst.reshape(()),
            nrm.reshape(()), mu)
